# baseline (speedup 1.0000x reference)
import functools

import jax
import jax.numpy as jnp
from jax import lax
from jax.experimental import pallas as pl
from jax.experimental.pallas import tpu as pltpu
from jax.experimental.pallas import tpu_sc as plsc

N = 10000
E = 320000
D = 128
G = 32
C = 10
EPS = 1e-5

NC = 2
NS = 16
NW = NC * NS
K = 128
CHUNKS = 80
CHUNKS2 = 160
E_PAD = NW * CHUNKS * K
DH = D // 2
PAD_ROWS = 8
RPT = 632
N_PAD = NS * RPT
RPT1 = 1024
N_PAD1 = NS * RPT1

BLK = 2000
NBLK = N // BLK


def _make_edge_kernel():
    mesh = plsc.VectorSubcoreMesh(core_axis_name="c", subcore_axis_name="s")

    nbuf = 4
    half = nbuf // 2

    @functools.partial(
        pl.kernel,
        out_type=jax.ShapeDtypeStruct((NC, N_PAD, DH), jnp.float32),
        mesh=mesh,
        scratch_types=[
            pltpu.VMEM((CHUNKS2, K), jnp.int32),
            pltpu.VMEM((CHUNKS2, K), jnp.int32),
            [pltpu.VMEM((K, DH), jnp.float32)] * nbuf,
            [pltpu.SemaphoreType.DMA] * nbuf,
            pltpu.VMEM_SHARED((N_PAD, DH), jnp.float32),
        ],
        compiler_params=pltpu.CompilerParams(use_tc_tiling_on_sc=False),
    )
    def edge_kernel(hs2, srcg, dstg, zblk, out, srcv, dstv, bufs, sems, acc):
        c = lax.axis_index("c")
        s = lax.axis_index("s")
        pltpu.sync_copy(srcg.at[c, s], srcv)
        pltpu.sync_copy(dstg.at[s], dstv)
        pltpu.sync_copy(zblk, acc.at[pl.ds(s * RPT, RPT)])
        plsc.subcore_barrier()

        for q in range(half):
            pltpu.async_copy(hs2.at[srcv.at[q]], bufs[q], sems[q])

        def body(i, carry):
            for q in range(nbuf):
                j = nbuf * i + q
                pltpu.make_async_copy(hs2.at[srcv.at[j]], bufs[q],
                                      sems[q]).wait()
                pltpu.async_copy(bufs[q], acc.at[dstv.at[j]], sems[q],
                                 add=True)
                p = (q + half) % nbuf
                jp = j + half

                def _wait_prev(p=p, j=j):
                    pltpu.make_async_copy(
                        bufs[p], acc.at[dstv.at[j - half]], sems[p]).wait()

                if q < half:
                    pl.when(i > 0)(_wait_prev)
                else:
                    _wait_prev()

                @pl.when(jp < CHUNKS2)
                def _(p=p, jp=jp):
                    pltpu.async_copy(hs2.at[srcv.at[jp]], bufs[p], sems[p])
            return carry

        lax.fori_loop(0, CHUNKS2 // nbuf, body, 0)
        for q in range(half, nbuf):
            pltpu.make_async_copy(
                bufs[q], acc.at[dstv.at[CHUNKS2 - nbuf + q]], sems[q]).wait()
        plsc.subcore_barrier()
        pltpu.sync_copy(acc.at[pl.ds(s * RPT, RPT)],
                        out.at[c, pl.ds(s * RPT, RPT)])

    return edge_kernel


def _make_deg_kernel():
    mesh = plsc.VectorSubcoreMesh(core_axis_name="c", subcore_axis_name="s")

    @functools.partial(
        pl.kernel,
        out_type=jax.ShapeDtypeStruct((NC * N_PAD1,), jnp.float32),
        mesh=mesh,
        scratch_types=[
            pltpu.VMEM((CHUNKS, K), jnp.int32),
            pltpu.VMEM((K,), jnp.float32),
            pltpu.VMEM_SHARED((N_PAD1,), jnp.float32),
        ],
        compiler_params=pltpu.CompilerParams(use_tc_tiling_on_sc=False),
    )
    def deg_kernel(dstg, z1, out, dstv, onesv, dacc):
        c = lax.axis_index("c")
        s = lax.axis_index("s")
        wid = c * NS + s
        pltpu.sync_copy(dstg.at[wid], dstv)
        for i in range(K // 16):
            onesv[pl.ds(16 * i, 16)] = jnp.full((16,), 1.0, jnp.float32)
        pltpu.sync_copy(z1, dacc.at[pl.ds(s * RPT1, RPT1)])
        plsc.subcore_barrier()

        def body(j, carry):
            pltpu.sync_copy(onesv, dacc.at[dstv.at[j]], add=True)
            return carry

        lax.fori_loop(0, CHUNKS, body, 0)
        plsc.subcore_barrier()
        pltpu.sync_copy(dacc.at[pl.ds(s * RPT1, RPT1)],
                        out.at[pl.ds(c * N_PAD1 + s * RPT1, RPT1)])

    return deg_kernel


_edge_call = _make_edge_kernel()
_deg_call = _make_deg_kernel()


def _mm_body(h_ref, w_ref, b_ref, d0_ref, d1_ref, hs_ref, selfb_ref):
    deg = d0_ref[...] + d1_ref[...] + 1.0
    inv = lax.rsqrt(deg)
    hw = jnp.dot(h_ref[...], w_ref[...], preferred_element_type=jnp.float32)
    hs_ref[...] = hw * inv
    selfb_ref[...] = hw * (inv * inv) + b_ref[...]


def _mm_call(h, w, b, d0, d1):
    return pl.pallas_call(
        _mm_body,
        grid=(NBLK,),
        in_specs=[
            pl.BlockSpec((BLK, D), lambda i: (i, 0)),
            pl.BlockSpec((D, D), lambda i: (0, 0)),
            pl.BlockSpec((1, D), lambda i: (0, 0)),
            pl.BlockSpec((BLK, 1), lambda i: (i, 0)),
            pl.BlockSpec((BLK, 1), lambda i: (i, 0)),
        ],
        out_specs=[
            pl.BlockSpec((BLK, D), lambda i: (i, 0)),
            pl.BlockSpec((BLK, D), lambda i: (i, 0)),
        ],
        out_shape=[
            jax.ShapeDtypeStruct((N, D), jnp.float32),
            jax.ShapeDtypeStruct((N, D), jnp.float32),
        ],
    )(h, w, b, d0, d1)


def _sum_body(a0_ref, a1_ref, selfb_ref, d0_ref, d1_ref, out_ref, st_ref):
    i = pl.program_id(0)
    deg = d0_ref[...] + d1_ref[...] + 1.0
    inv = lax.rsqrt(deg)
    a0 = a0_ref[...].reshape(BLK, DH)
    a1 = a1_ref[...].reshape(BLK, DH)
    o = jnp.concatenate([a0, a1], axis=1) * inv + selfb_ref[...]
    out_ref[...] = o

    @pl.when(i == 0)
    def _():
        st_ref[...] = jnp.zeros_like(st_ref)

    s1 = jnp.sum(o, axis=0, keepdims=True)
    s2 = jnp.sum(o * o, axis=0, keepdims=True)
    st_ref[...] += jnp.concatenate([s1, s2], axis=0)


def _sum_call(accp, selfb, d0, d1):
    return pl.pallas_call(
        _sum_body,
        grid=(NBLK,),
        in_specs=[
            pl.BlockSpec((1, BLK, DH), lambda i: (0, i, 0)),
            pl.BlockSpec((1, BLK, DH), lambda i: (1, i, 0)),
            pl.BlockSpec((BLK, D), lambda i: (i, 0)),
            pl.BlockSpec((BLK, 1), lambda i: (i, 0)),
            pl.BlockSpec((BLK, 1), lambda i: (i, 0)),
        ],
        out_specs=[
            pl.BlockSpec((BLK, D), lambda i: (i, 0)),
            pl.BlockSpec((2, D), lambda i: (0, 0)),
        ],
        out_shape=[
            jax.ShapeDtypeStruct((N, D), jnp.float32),
            jax.ShapeDtypeStruct((2, D), jnp.float32),
        ],
    )(accp, accp, selfb, d0, d1)


def _bn_from_stats(o, st, g, be):
    mean = st[0:1, :] * (1.0 / N)
    var = st[1:2, :] * (1.0 / N) - mean * mean
    scale = lax.rsqrt(var + EPS) * g
    return jnp.maximum((o - mean) * scale + be, 0.0)


def _pool_accumulate(i, h, br_ref, bc_ref, mx_ref, sm_ref, cnt_ref):
    neg_inf = jnp.float32(-jnp.inf)

    @pl.when(i == 0)
    def _():
        mx_ref[...] = jnp.full((G, D), neg_inf, jnp.float32)
        sm_ref[...] = jnp.zeros((G, D), jnp.float32)
        if cnt_ref is not None:
            cnt_ref[...] = jnp.zeros((G, 1), jnp.float32)

    bc = bc_ref[...].reshape(1, BLK)
    masks = (lax.broadcasted_iota(jnp.int32, (G, BLK), 0) == bc
             ).astype(jnp.float32)
    sm_ref[...] += jnp.dot(masks, h, preferred_element_type=jnp.float32)
    if cnt_ref is not None:
        cnt_ref[...] += jnp.sum(masks, axis=1, keepdims=True)
    br = br_ref[...]
    bmin = jnp.min(br)
    bmax = jnp.max(br)
    for gi in range(G):
        @pl.when((bmin <= gi) & (gi <= bmax))
        def _(gi=gi):
            m = br == gi
            blk_max = jnp.max(jnp.where(m, h, neg_inf), axis=0,
                              keepdims=True)
            mx_ref[gi:gi + 1, :] = jnp.maximum(mx_ref[gi:gi + 1, :], blk_max)


def _bn_pool_mm_body(o_ref, st_ref, g_ref, be_ref, br_ref, bc_ref, w_ref,
                     b_ref, d0_ref, d1_ref, hs_ref, selfb_ref, mx_ref,
                     sm_ref, cnt_ref):
    i = pl.program_id(0)
    h = _bn_from_stats(o_ref[...], st_ref[...], g_ref[...], be_ref[...])
    _pool_accumulate(i, h, br_ref, bc_ref, mx_ref, sm_ref, cnt_ref)
    deg = d0_ref[...] + d1_ref[...] + 1.0
    inv = lax.rsqrt(deg)
    hw = jnp.dot(h, w_ref[...], preferred_element_type=jnp.float32)
    hs_ref[...] = hw * inv
    selfb_ref[...] = hw * (inv * inv) + b_ref[...]


def _bn_pool_mm_call(o, st, g, be, br, bc, w, b, d0, d1):
    return pl.pallas_call(
        _bn_pool_mm_body,
        grid=(NBLK,),
        in_specs=[
            pl.BlockSpec((BLK, D), lambda i: (i, 0)),
            pl.BlockSpec((2, D), lambda i: (0, 0)),
            pl.BlockSpec((1, D), lambda i: (0, 0)),
            pl.BlockSpec((1, D), lambda i: (0, 0)),
            pl.BlockSpec((BLK, 1), lambda i: (i, 0)),
            pl.BlockSpec((1, 1, BLK), lambda i: (i, 0, 0)),
            pl.BlockSpec((D, D), lambda i: (0, 0)),
            pl.BlockSpec((1, D), lambda i: (0, 0)),
            pl.BlockSpec((BLK, 1), lambda i: (i, 0)),
            pl.BlockSpec((BLK, 1), lambda i: (i, 0)),
        ],
        out_specs=[
            pl.BlockSpec((BLK, D), lambda i: (i, 0)),
            pl.BlockSpec((BLK, D), lambda i: (i, 0)),
            pl.BlockSpec((G, D), lambda i: (0, 0)),
            pl.BlockSpec((G, D), lambda i: (0, 0)),
            pl.BlockSpec((G, 1), lambda i: (0, 0)),
        ],
        out_shape=[
            jax.ShapeDtypeStruct((N, D), jnp.float32),
            jax.ShapeDtypeStruct((N, D), jnp.float32),
            jax.ShapeDtypeStruct((G, D), jnp.float32),
            jax.ShapeDtypeStruct((G, D), jnp.float32),
            jax.ShapeDtypeStruct((G, 1), jnp.float32),
        ],
    )(o, st, g, be, br, bc, w, b, d0, d1)


def _pool_only_body(o_ref, st_ref, g_ref, be_ref, br_ref, bc_ref,
                    mx_ref, sm_ref):
    i = pl.program_id(0)
    h = _bn_from_stats(o_ref[...], st_ref[...], g_ref[...], be_ref[...])
    _pool_accumulate(i, h, br_ref, bc_ref, mx_ref, sm_ref, None)


def _pool_only_call(o, st, g, be, br, bc):
    return pl.pallas_call(
        _pool_only_body,
        grid=(NBLK,),
        in_specs=[
            pl.BlockSpec((BLK, D), lambda i: (i, 0)),
            pl.BlockSpec((2, D), lambda i: (0, 0)),
            pl.BlockSpec((1, D), lambda i: (0, 0)),
            pl.BlockSpec((1, D), lambda i: (0, 0)),
            pl.BlockSpec((BLK, 1), lambda i: (i, 0)),
            pl.BlockSpec((1, 1, BLK), lambda i: (i, 0, 0)),
        ],
        out_specs=[
            pl.BlockSpec((G, D), lambda i: (0, 0)),
            pl.BlockSpec((G, D), lambda i: (0, 0)),
        ],
        out_shape=[
            jax.ShapeDtypeStruct((G, D), jnp.float32),
            jax.ShapeDtypeStruct((G, D), jnp.float32),
        ],
    )(o, st, g, be, br, bc)


def _fin_body(mx0_ref, sm0_ref, mx1_ref, sm1_ref, cnt_ref, wl_ref, bl_ref,
              out_ref):
    c = jnp.maximum(cnt_ref[...], 1.0)
    mean = (sm0_ref[...] + sm1_ref[...]) / c
    mxs = mx0_ref[...] + mx1_ref[...]
    acc = jnp.concatenate([mxs, mean], axis=1)
    out_ref[...] = jnp.dot(acc, wl_ref[...],
                           preferred_element_type=jnp.float32) + bl_ref[...]


def _fin_call(mx0, sm0, mx1, sm1, cnt, wlin, blin):
    return pl.pallas_call(
        _fin_body,
        out_shape=jax.ShapeDtypeStruct((G, C), jnp.float32),
    )(mx0, sm0, mx1, sm1, cnt, wlin, blin)


def kernel(x, edge_index, batch, W0, b0, W1, b1, W2, b2,
           g0, be0, g1, be1, g2, be2, Wlin, blin):
    src = edge_index[0]
    dst = edge_index[1]
    pad = E_PAD - E
    ar = jnp.arange(pad, dtype=jnp.int32)
    src_p = jnp.concatenate([src, ar])
    dst_p = jnp.concatenate([dst, N + (ar & (PAD_ROWS - 1))])
    srcg = jnp.stack([2 * src_p, 2 * src_p + 1]).reshape(NC, NS, CHUNKS2, K)
    dstg = dst_p.reshape(NS, CHUNKS2, K)
    dstgd = dstg.reshape(NW, CHUNKS, K)

    zblk = jnp.zeros((RPT, DH), jnp.float32)
    z1 = jnp.zeros((RPT1,), jnp.float32)

    degf = _deg_call(dstgd, z1)
    d0 = degf[:N].reshape(N, 1)
    d1 = degf[N_PAD1:N_PAD1 + N].reshape(N, 1)

    br = batch.reshape(N, 1)
    bc = batch.reshape(NBLK, 1, BLK)

    hs0, selfb0 = _mm_call(x, W0, b0.reshape(1, D), d0, d1)
    acc0 = _edge_call(hs0.reshape(2 * N, DH), srcg, dstg, zblk)
    o0, st0 = _sum_call(acc0, selfb0, d0, d1)
    hs1, selfb1, mx0, sm0, cnt = _bn_pool_mm_call(
        o0, st0, g0.reshape(1, D), be0.reshape(1, D), br, bc,
        W1, b1.reshape(1, D), d0, d1)
    acc1 = _edge_call(hs1.reshape(2 * N, DH), srcg, dstg, zblk)
    o1, st1 = _sum_call(acc1, selfb1, d0, d1)
    mx1, sm1 = _pool_only_call(o1, st1, g1.reshape(1, D),
                               be1.reshape(1, D), br, bc)
    return _fin_call(mx0, sm0, mx1, sm1, cnt, Wlin, blin.reshape(1, C))


# speedup vs baseline: 1.0107x; 1.0107x over previous
import functools

import jax
import jax.numpy as jnp
from jax import lax
from jax.experimental import pallas as pl
from jax.experimental.pallas import tpu as pltpu
from jax.experimental.pallas import tpu_sc as plsc

N = 10000
E = 320000
D = 128
G = 32
C = 10
EPS = 1e-5

NC = 2
NS = 16
NW = NC * NS
K = 128
CHUNKS = 80
CHUNKS2 = 160
E_PAD = NW * CHUNKS * K
DH = D // 2
PAD_ROWS = 8
RPT = 632
N_PAD = NS * RPT
RPT1 = 1024
N_PAD1 = NS * RPT1

BLK = 1000
NBLK = N // BLK


def _make_edge_kernel():
    mesh = plsc.VectorSubcoreMesh(core_axis_name="c", subcore_axis_name="s")

    nbuf = 4
    half = nbuf // 2

    @functools.partial(
        pl.kernel,
        out_type=jax.ShapeDtypeStruct((NC, N_PAD, DH), jnp.float32),
        mesh=mesh,
        scratch_types=[
            pltpu.VMEM((CHUNKS2, K), jnp.int32),
            pltpu.VMEM((CHUNKS2, K), jnp.int32),
            [pltpu.VMEM((K, DH), jnp.float32)] * nbuf,
            [pltpu.SemaphoreType.DMA] * nbuf,
            pltpu.VMEM_SHARED((N_PAD, DH), jnp.float32),
        ],
        compiler_params=pltpu.CompilerParams(use_tc_tiling_on_sc=False),
    )
    def edge_kernel(hs2, srcg, dstg, zblk, out, srcv, dstv, bufs, sems, acc):
        c = lax.axis_index("c")
        s = lax.axis_index("s")
        pltpu.sync_copy(srcg.at[c, s], srcv)
        pltpu.sync_copy(dstg.at[s], dstv)
        pltpu.sync_copy(zblk, acc.at[pl.ds(s * RPT, RPT)])
        plsc.subcore_barrier()

        for q in range(half):
            pltpu.async_copy(hs2.at[srcv.at[q]], bufs[q], sems[q])

        def body(i, carry):
            for q in range(nbuf):
                j = nbuf * i + q
                pltpu.make_async_copy(hs2.at[srcv.at[j]], bufs[q],
                                      sems[q]).wait()
                pltpu.async_copy(bufs[q], acc.at[dstv.at[j]], sems[q],
                                 add=True)
                p = (q + half) % nbuf
                jp = j + half

                def _wait_prev(p=p, j=j):
                    pltpu.make_async_copy(
                        bufs[p], acc.at[dstv.at[j - half]], sems[p]).wait()

                if q < half:
                    pl.when(i > 0)(_wait_prev)
                else:
                    _wait_prev()

                @pl.when(jp < CHUNKS2)
                def _(p=p, jp=jp):
                    pltpu.async_copy(hs2.at[srcv.at[jp]], bufs[p], sems[p])
            return carry

        lax.fori_loop(0, CHUNKS2 // nbuf, body, 0)
        for q in range(half, nbuf):
            pltpu.make_async_copy(
                bufs[q], acc.at[dstv.at[CHUNKS2 - nbuf + q]], sems[q]).wait()
        plsc.subcore_barrier()
        pltpu.sync_copy(acc.at[pl.ds(s * RPT, RPT)],
                        out.at[c, pl.ds(s * RPT, RPT)])

    return edge_kernel


def _make_deg_kernel():
    mesh = plsc.VectorSubcoreMesh(core_axis_name="c", subcore_axis_name="s")

    @functools.partial(
        pl.kernel,
        out_type=jax.ShapeDtypeStruct((NC * N_PAD1,), jnp.float32),
        mesh=mesh,
        scratch_types=[
            pltpu.VMEM((CHUNKS, K), jnp.int32),
            pltpu.VMEM((K,), jnp.float32),
            pltpu.VMEM_SHARED((N_PAD1,), jnp.float32),
        ],
        compiler_params=pltpu.CompilerParams(use_tc_tiling_on_sc=False),
    )
    def deg_kernel(dstg, z1, out, dstv, onesv, dacc):
        c = lax.axis_index("c")
        s = lax.axis_index("s")
        wid = c * NS + s
        pltpu.sync_copy(dstg.at[wid], dstv)
        for i in range(K // 16):
            onesv[pl.ds(16 * i, 16)] = jnp.full((16,), 1.0, jnp.float32)
        pltpu.sync_copy(z1, dacc.at[pl.ds(s * RPT1, RPT1)])
        plsc.subcore_barrier()

        def body(j, carry):
            pltpu.sync_copy(onesv, dacc.at[dstv.at[j]], add=True)
            return carry

        lax.fori_loop(0, CHUNKS, body, 0)
        plsc.subcore_barrier()
        pltpu.sync_copy(dacc.at[pl.ds(s * RPT1, RPT1)],
                        out.at[pl.ds(c * N_PAD1 + s * RPT1, RPT1)])

    return deg_kernel


_edge_call = _make_edge_kernel()
_deg_call = _make_deg_kernel()


def _mm_body(h_ref, w_ref, b_ref, d0_ref, d1_ref, hs_ref, selfb_ref):
    deg = d0_ref[...] + d1_ref[...] + 1.0
    inv = lax.rsqrt(deg)
    hw = jnp.dot(h_ref[...], w_ref[...], preferred_element_type=jnp.float32)
    hs_ref[...] = hw * inv
    selfb_ref[...] = hw * (inv * inv) + b_ref[...]


def _mm_call(h, w, b, d0, d1):
    return pl.pallas_call(
        _mm_body,
        grid=(NBLK,),
        in_specs=[
            pl.BlockSpec((BLK, D), lambda i: (i, 0)),
            pl.BlockSpec((D, D), lambda i: (0, 0)),
            pl.BlockSpec((1, D), lambda i: (0, 0)),
            pl.BlockSpec((BLK, 1), lambda i: (i, 0)),
            pl.BlockSpec((BLK, 1), lambda i: (i, 0)),
        ],
        out_specs=[
            pl.BlockSpec((BLK, D), lambda i: (i, 0)),
            pl.BlockSpec((BLK, D), lambda i: (i, 0)),
        ],
        out_shape=[
            jax.ShapeDtypeStruct((N, D), jnp.float32),
            jax.ShapeDtypeStruct((N, D), jnp.float32),
        ],
    )(h, w, b, d0, d1)


def _sum_body(a0_ref, a1_ref, selfb_ref, d0_ref, d1_ref, out_ref, st_ref):
    i = pl.program_id(0)
    deg = d0_ref[...] + d1_ref[...] + 1.0
    inv = lax.rsqrt(deg)
    a0 = a0_ref[...].reshape(BLK, DH)
    a1 = a1_ref[...].reshape(BLK, DH)
    o = jnp.concatenate([a0, a1], axis=1) * inv + selfb_ref[...]
    out_ref[...] = o

    @pl.when(i == 0)
    def _():
        st_ref[...] = jnp.zeros_like(st_ref)

    s1 = jnp.sum(o, axis=0, keepdims=True)
    s2 = jnp.sum(o * o, axis=0, keepdims=True)
    st_ref[...] += jnp.concatenate([s1, s2], axis=0)


def _sum_call(accp, selfb, d0, d1):
    return pl.pallas_call(
        _sum_body,
        grid=(NBLK,),
        in_specs=[
            pl.BlockSpec((1, BLK, DH), lambda i: (0, i, 0)),
            pl.BlockSpec((1, BLK, DH), lambda i: (1, i, 0)),
            pl.BlockSpec((BLK, D), lambda i: (i, 0)),
            pl.BlockSpec((BLK, 1), lambda i: (i, 0)),
            pl.BlockSpec((BLK, 1), lambda i: (i, 0)),
        ],
        out_specs=[
            pl.BlockSpec((BLK, D), lambda i: (i, 0)),
            pl.BlockSpec((2, D), lambda i: (0, 0)),
        ],
        out_shape=[
            jax.ShapeDtypeStruct((N, D), jnp.float32),
            jax.ShapeDtypeStruct((2, D), jnp.float32),
        ],
    )(accp, accp, selfb, d0, d1)


def _bn_from_stats(o, st, g, be):
    mean = st[0:1, :] * (1.0 / N)
    var = st[1:2, :] * (1.0 / N) - mean * mean
    scale = lax.rsqrt(var + EPS) * g
    return jnp.maximum((o - mean) * scale + be, 0.0)


def _pool_accumulate(i, h, br_ref, bc_ref, mx_ref, sm_ref, cnt_ref):
    neg_inf = jnp.float32(-jnp.inf)

    @pl.when(i == 0)
    def _():
        mx_ref[...] = jnp.full((G, D), neg_inf, jnp.float32)
        sm_ref[...] = jnp.zeros((G, D), jnp.float32)
        if cnt_ref is not None:
            cnt_ref[...] = jnp.zeros((G, 1), jnp.float32)

    bc = bc_ref[...].reshape(1, BLK)
    masks = (lax.broadcasted_iota(jnp.int32, (G, BLK), 0) == bc
             ).astype(jnp.float32)
    sm_ref[...] += jnp.dot(masks, h, preferred_element_type=jnp.float32)
    if cnt_ref is not None:
        cnt_ref[...] += jnp.sum(masks, axis=1, keepdims=True)
    br = br_ref[...]
    bmin = jnp.min(br)
    bmax = jnp.max(br)
    for gi in range(G):
        @pl.when((bmin <= gi) & (gi <= bmax))
        def _(gi=gi):
            m = br == gi
            blk_max = jnp.max(jnp.where(m, h, neg_inf), axis=0,
                              keepdims=True)
            mx_ref[gi:gi + 1, :] = jnp.maximum(mx_ref[gi:gi + 1, :], blk_max)


def _bn_pool_mm_body(o_ref, st_ref, g_ref, be_ref, br_ref, bc_ref, w_ref,
                     b_ref, d0_ref, d1_ref, hs_ref, selfb_ref, mx_ref,
                     sm_ref, cnt_ref):
    i = pl.program_id(0)
    h = _bn_from_stats(o_ref[...], st_ref[...], g_ref[...], be_ref[...])
    _pool_accumulate(i, h, br_ref, bc_ref, mx_ref, sm_ref, cnt_ref)
    deg = d0_ref[...] + d1_ref[...] + 1.0
    inv = lax.rsqrt(deg)
    hw = jnp.dot(h, w_ref[...], preferred_element_type=jnp.float32)
    hs_ref[...] = hw * inv
    selfb_ref[...] = hw * (inv * inv) + b_ref[...]


def _bn_pool_mm_call(o, st, g, be, br, bc, w, b, d0, d1):
    return pl.pallas_call(
        _bn_pool_mm_body,
        grid=(NBLK,),
        in_specs=[
            pl.BlockSpec((BLK, D), lambda i: (i, 0)),
            pl.BlockSpec((2, D), lambda i: (0, 0)),
            pl.BlockSpec((1, D), lambda i: (0, 0)),
            pl.BlockSpec((1, D), lambda i: (0, 0)),
            pl.BlockSpec((BLK, 1), lambda i: (i, 0)),
            pl.BlockSpec((1, 1, BLK), lambda i: (i, 0, 0)),
            pl.BlockSpec((D, D), lambda i: (0, 0)),
            pl.BlockSpec((1, D), lambda i: (0, 0)),
            pl.BlockSpec((BLK, 1), lambda i: (i, 0)),
            pl.BlockSpec((BLK, 1), lambda i: (i, 0)),
        ],
        out_specs=[
            pl.BlockSpec((BLK, D), lambda i: (i, 0)),
            pl.BlockSpec((BLK, D), lambda i: (i, 0)),
            pl.BlockSpec((G, D), lambda i: (0, 0)),
            pl.BlockSpec((G, D), lambda i: (0, 0)),
            pl.BlockSpec((G, 1), lambda i: (0, 0)),
        ],
        out_shape=[
            jax.ShapeDtypeStruct((N, D), jnp.float32),
            jax.ShapeDtypeStruct((N, D), jnp.float32),
            jax.ShapeDtypeStruct((G, D), jnp.float32),
            jax.ShapeDtypeStruct((G, D), jnp.float32),
            jax.ShapeDtypeStruct((G, 1), jnp.float32),
        ],
    )(o, st, g, be, br, bc, w, b, d0, d1)


def _pool_only_body(o_ref, st_ref, g_ref, be_ref, br_ref, bc_ref,
                    mx_ref, sm_ref):
    i = pl.program_id(0)
    h = _bn_from_stats(o_ref[...], st_ref[...], g_ref[...], be_ref[...])
    _pool_accumulate(i, h, br_ref, bc_ref, mx_ref, sm_ref, None)


def _pool_only_call(o, st, g, be, br, bc):
    return pl.pallas_call(
        _pool_only_body,
        grid=(NBLK,),
        in_specs=[
            pl.BlockSpec((BLK, D), lambda i: (i, 0)),
            pl.BlockSpec((2, D), lambda i: (0, 0)),
            pl.BlockSpec((1, D), lambda i: (0, 0)),
            pl.BlockSpec((1, D), lambda i: (0, 0)),
            pl.BlockSpec((BLK, 1), lambda i: (i, 0)),
            pl.BlockSpec((1, 1, BLK), lambda i: (i, 0, 0)),
        ],
        out_specs=[
            pl.BlockSpec((G, D), lambda i: (0, 0)),
            pl.BlockSpec((G, D), lambda i: (0, 0)),
        ],
        out_shape=[
            jax.ShapeDtypeStruct((G, D), jnp.float32),
            jax.ShapeDtypeStruct((G, D), jnp.float32),
        ],
    )(o, st, g, be, br, bc)


def _fin_body(mx0_ref, sm0_ref, mx1_ref, sm1_ref, cnt_ref, wl_ref, bl_ref,
              out_ref):
    c = jnp.maximum(cnt_ref[...], 1.0)
    mean = (sm0_ref[...] + sm1_ref[...]) / c
    mxs = mx0_ref[...] + mx1_ref[...]
    acc = jnp.concatenate([mxs, mean], axis=1)
    out_ref[...] = jnp.dot(acc, wl_ref[...],
                           preferred_element_type=jnp.float32) + bl_ref[...]


def _fin_call(mx0, sm0, mx1, sm1, cnt, wlin, blin):
    return pl.pallas_call(
        _fin_body,
        out_shape=jax.ShapeDtypeStruct((G, C), jnp.float32),
    )(mx0, sm0, mx1, sm1, cnt, wlin, blin)


def kernel(x, edge_index, batch, W0, b0, W1, b1, W2, b2,
           g0, be0, g1, be1, g2, be2, Wlin, blin):
    src = edge_index[0]
    dst = edge_index[1]
    pad = E_PAD - E
    ar = jnp.arange(pad, dtype=jnp.int32)
    src_p = jnp.concatenate([src, ar])
    dst_p = jnp.concatenate([dst, N + (ar & (PAD_ROWS - 1))])
    srcg = jnp.stack([2 * src_p, 2 * src_p + 1]).reshape(NC, NS, CHUNKS2, K)
    dstg = dst_p.reshape(NS, CHUNKS2, K)
    dstgd = dstg.reshape(NW, CHUNKS, K)

    zblk = jnp.zeros((RPT, DH), jnp.float32)
    z1 = jnp.zeros((RPT1,), jnp.float32)

    degf = _deg_call(dstgd, z1)
    d0 = degf[:N].reshape(N, 1)
    d1 = degf[N_PAD1:N_PAD1 + N].reshape(N, 1)

    br = batch.reshape(N, 1)
    bc = batch.reshape(NBLK, 1, BLK)

    hs0, selfb0 = _mm_call(x, W0, b0.reshape(1, D), d0, d1)
    acc0 = _edge_call(hs0.reshape(2 * N, DH), srcg, dstg, zblk)
    o0, st0 = _sum_call(acc0, selfb0, d0, d1)
    hs1, selfb1, mx0, sm0, cnt = _bn_pool_mm_call(
        o0, st0, g0.reshape(1, D), be0.reshape(1, D), br, bc,
        W1, b1.reshape(1, D), d0, d1)
    acc1 = _edge_call(hs1.reshape(2 * N, DH), srcg, dstg, zblk)
    o1, st1 = _sum_call(acc1, selfb1, d0, d1)
    mx1, sm1 = _pool_only_call(o1, st1, g1.reshape(1, D),
                               be1.reshape(1, D), br, bc)
    return _fin_call(mx0, sm0, mx1, sm1, cnt, Wlin, blin.reshape(1, C))
